# baseline (speedup 1.0000x reference)
import functools

import jax
import jax.numpy as jnp
from jax import lax
from jax.experimental import pallas as pl
from jax.experimental.pallas import tpu as pltpu
from jax.experimental.pallas import tpu_sc as plsc

_M = 10000
_F = 128
_FH = _F // 2
_E = 320000
_NT = 16
_EPT = _E // _NT
_C = 80
_NCH = _EPT // _C
_NZC = _M // _C
_LGH = _FH // 16


def _spmv_body(h_hbm, ed_hbm, out_hbm, *refs):
    ed = refs[0:8]
    rows = refs[8:12]
    acc_sh = refs[12]
    sem_g = refs[13:17]
    sem_s = refs[17:21]
    sem_e = refs[21:29]
    c = lax.axis_index("c")
    s = lax.axis_index("s")

    r0 = rows[0]

    def _zrow(r, carry):
        for j in range(_LGH):
            r0[r, pl.ds(j * 16, 16)] = jnp.zeros((16,), jnp.float32)
        return carry
    lax.fori_loop(0, _C, _zrow, 0)
    for rep in range(-(-_NZC // _NT)):
        cid = s + _NT * rep

        @pl.when(cid < _NZC)
        def _():
            pltpu.sync_copy(r0, acc_sh.at[pl.ds(cid * _C, _C)])
    plsc.subcore_barrier()

    def _fire_edata(t, e):
        pltpu.async_copy(ed_hbm.at[c, s, t], ed[e], sem_e[e])

    def _wait_edata(t, e):
        pltpu.make_async_copy(ed_hbm.at[c, s, t], ed[e], sem_e[e]).wait()

    def _gather(t, b, e):
        pltpu.async_copy(h_hbm.at[ed[e].at[0]], rows[b], sem_g[b])

    def _wait_gather(t, b, e):
        pltpu.make_async_copy(h_hbm.at[ed[e].at[0]], rows[b], sem_g[b]).wait()

    def _scatter(t, b, e):
        pltpu.async_copy(rows[b], acc_sh.at[ed[e].at[1]], sem_s[b], add=True)

    def _wait_scatter(t, b, e):
        pltpu.make_async_copy(rows[b], acc_sh.at[ed[e].at[1]], sem_s[b]).wait()

    def _scale(b, e):
        rb = rows[b]
        eb = ed[e]

        def _e16(g, carry):
            w16 = lax.bitcast_convert_type(eb[2, pl.ds(g * 16, 16)], jnp.float32)
            for i in range(16):
                wv = lax.gather(
                    w16, jnp.full((16, 1), i, jnp.int32),
                    dimension_numbers=lax.GatherDimensionNumbers(
                        offset_dims=(), collapsed_slice_dims=(0,),
                        start_index_map=(0,)),
                    slice_sizes=(1,),
                    mode=lax.GatherScatterMode.PROMISE_IN_BOUNDS)
                for j in range(_LGH):
                    sl = pl.ds(j * 16, 16)
                    rb[g * 16 + i, sl] = rb[g * 16 + i, sl] * wv
            return carry
        lax.fori_loop(0, _C // 16, _e16, 0)

    for t in range(6):
        _fire_edata(t, t)
    _wait_edata(0, 0)
    _gather(0, 0, 0)
    _wait_edata(1, 1)
    _gather(1, 1, 1)

    def _substep(t, b, e):
        bf = (b + 2) % 4

        @pl.when(t + 2 < _NCH)
        def _():
            @pl.when(t >= 2)
            def _():
                _wait_scatter(t - 2, bf, (e + 6) % 8)

            @pl.when(t + 6 < _NCH)
            def _():
                _fire_edata(t + 6, (e + 6) % 8)
            _wait_edata(t + 2, (e + 2) % 8)
            _gather(t + 2, bf, (e + 2) % 8)
        _wait_gather(t, b, e)
        _scale(b, e)
        _scatter(t, b, e)

    def _outer(i, carry):
        for b8 in range(8):
            _substep(i * 8 + b8, b8 % 4, b8)
        return carry
    lax.fori_loop(0, _NCH // 8, _outer, 0)

    for t in range((_NCH // 8) * 8, _NCH):
        _substep(t, t % 4, t % 8)
    for t in range(_NCH - 4, _NCH):
        _wait_scatter(t, t % 4, t % 8)

    plsc.subcore_barrier()
    for rep in range(-(-_NZC // _NT)):
        cid = s + _NT * rep

        @pl.when(cid < _NZC)
        def _():
            o = cid * _C
            pltpu.sync_copy(acc_sh.at[pl.ds(o, _C)], out_hbm.at[c, pl.ds(o, _C)])


@functools.lru_cache(maxsize=None)
def _make_spmv():
    scratch = (
        [pltpu.VMEM((3, _C), jnp.int32) for _ in range(8)]
        + [pltpu.VMEM((_C, _FH), jnp.float32) for _ in range(4)]
        + [pltpu.VMEM_SHARED((_M, _FH), jnp.float32)]
        + [pltpu.SemaphoreType.DMA] * 16
    )
    return pl.kernel(
        _spmv_body,
        out_type=jax.ShapeDtypeStruct((2, _M, _FH), jnp.float32),
        scratch_types=scratch,
        mesh=plsc.VectorSubcoreMesh(core_axis_name="c", subcore_axis_name="s"),
        compiler_params=pltpu.CompilerParams(use_tc_tiling_on_sc=False),
    )


def _dense_body(x0_ref, s1_ref, s2_ref, s3_ref, dk_ref, pk_ref, o_ref):
    dk = dk_ref[...]
    pkt = pk_ref[...].T
    w0 = pkt * dk[:, 0, 0][:, None]
    w1 = pkt * dk[:, 0, 1][:, None]
    w2 = pkt * dk[:, 0, 2][:, None]
    w3 = pkt * dk[:, 0, 3][:, None]
    v0 = w0 - w2
    v1 = w1 - 3.0 * w3
    v2 = 2.0 * w2
    v3 = 4.0 * w3
    acc = jnp.zeros_like(o_ref)
    for h in range(2):
        lo, hi = h * _FH, (h + 1) * _FH
        acc += jnp.dot(x0_ref[h], v0[lo:hi, :], preferred_element_type=jnp.float32)
        acc += jnp.dot(s1_ref[h], v1[lo:hi, :], preferred_element_type=jnp.float32)
        acc += jnp.dot(s2_ref[h], v2[lo:hi, :], preferred_element_type=jnp.float32)
        acc += jnp.dot(s3_ref[h], v3[lo:hi, :], preferred_element_type=jnp.float32)
    o_ref[...] = acc


def _dense(x0h, s1, s2, s3, dkernel, pkernel):
    bm = _M // 10
    hspec = pl.BlockSpec((2, bm, _FH), lambda i: (0, i, 0))
    return pl.pallas_call(
        _dense_body,
        grid=(10,),
        in_specs=[
            hspec,
            hspec,
            hspec,
            hspec,
            pl.BlockSpec((_F, 1, 4), lambda i: (0, 0, 0)),
            pl.BlockSpec((_F, _F), lambda i: (0, 0)),
        ],
        out_specs=pl.BlockSpec((bm, _F), lambda i: (i, 0)),
        out_shape=jax.ShapeDtypeStruct((_M, _F), jnp.float32),
    )(x0h, s1, s2, s3, dkernel, pkernel)


def kernel(x, edge_index, edge_weight, dkernel, pkernel):
    m = x.shape[1]
    x0 = jnp.transpose(x, (1, 0, 2)).reshape(m, -1)
    x0h = jnp.transpose(x0.reshape(m, 2, _FH), (1, 0, 2))

    src = edge_index[0].astype(jnp.int32).reshape(_NT, _NCH, _C)
    dst = edge_index[1].astype(jnp.int32).reshape(_NT, _NCH, _C)
    wbits = lax.bitcast_convert_type(
        edge_weight.astype(jnp.float32), jnp.int32).reshape(_NT, _NCH, _C)
    src2 = jnp.stack([src, src + m], axis=0)
    dst2 = jnp.broadcast_to(dst, (2, _NT, _NCH, _C))
    w2 = jnp.broadcast_to(wbits, (2, _NT, _NCH, _C))
    ed = jnp.stack([src2, dst2, w2], axis=3)

    spmv = _make_spmv()
    s1 = spmv(x0h.reshape(2 * m, _FH), ed)
    s2 = spmv(s1.reshape(2 * m, _FH), ed)
    s3 = spmv(s2.reshape(2 * m, _FH), ed)

    out = _dense(x0h, s1, s2, s3, dkernel, pkernel)
    return out.reshape(1, m, -1)


# speedup vs baseline: 2.3933x; 2.3933x over previous
import functools

import jax
import jax.numpy as jnp
from jax import lax
from jax.experimental import pallas as pl
from jax.experimental.pallas import tpu as pltpu
from jax.experimental.pallas import tpu_sc as plsc

_M = 10000
_F = 128
_E = 320000
_NW = 32
_EPW = _E // _NW
_C = 80
_NCHUNK = _EPW // _C
_RC = 200
_NRC = _M // _RC
_RREP = -(-_NRC // 16)
_LG = _F // 16


def _spmv_body(combine, *args):
    if combine:
        p_hbm, ed_hbm, out_hbm, s_hbm = args[0:4]
        refs = args[4:]
        h_hbm = s_hbm
    else:
        h_hbm, ed_hbm, out_hbm = args[0:3]
        refs = args[3:]
    ed = refs[0:8]
    rows = refs[8:12]
    acc_sh = refs[12]
    sem_g = refs[13:17]
    sem_s = refs[17:21]
    sem_e = refs[21:29]
    c = lax.axis_index("c")
    s = lax.axis_index("s")
    wid = c * 16 + s

    r0, r1, r2 = rows[0], rows[1], rows[2]
    if combine:
        for rep in range(-(-_NCHUNK // 16)):
            cid = s + 16 * rep

            @pl.when(cid < _NCHUNK)
            def _():
                o = cid * _C
                pltpu.sync_copy(p_hbm.at[0, pl.ds(o, _C)], r1)
                pltpu.sync_copy(p_hbm.at[1, pl.ds(o, _C)], r2)

                def _crow(r, carry):
                    for j in range(_LG):
                        sl = pl.ds(j * 16, 16)
                        r1[r, sl] = r1[r, sl] + r2[r, sl]
                    return carry
                lax.fori_loop(0, _C, _crow, 0)
                pltpu.sync_copy(r1, s_hbm.at[pl.ds(o, _C)])

    def _zrow(r, carry):
        for j in range(_LG):
            r0[r, pl.ds(j * 16, 16)] = jnp.zeros((16,), jnp.float32)
        return carry
    lax.fori_loop(0, _C, _zrow, 0)
    for rep in range(-(-_NCHUNK // 16)):
        cid = s + 16 * rep

        @pl.when(cid < _NCHUNK)
        def _():
            pltpu.sync_copy(r0, acc_sh.at[pl.ds(cid * _C, _C)])
    plsc.subcore_barrier()

    def _fire_edata(t, e):
        pltpu.async_copy(ed_hbm.at[wid, t], ed[e], sem_e[e])

    def _wait_edata(t, e):
        pltpu.make_async_copy(ed_hbm.at[wid, t], ed[e], sem_e[e]).wait()

    def _gather(t, b, e):
        pltpu.async_copy(h_hbm.at[ed[e].at[0]], rows[b], sem_g[b])

    def _wait_gather(t, b, e):
        pltpu.make_async_copy(h_hbm.at[ed[e].at[0]], rows[b], sem_g[b]).wait()

    def _scatter(t, b, e):
        pltpu.async_copy(rows[b], acc_sh.at[ed[e].at[1]], sem_s[b], add=True)

    def _wait_scatter(t, b, e):
        pltpu.make_async_copy(rows[b], acc_sh.at[ed[e].at[1]], sem_s[b]).wait()

    def _scale(b, e):
        rb = rows[b]
        eb = ed[e]

        def _e16(g, carry):
            w16 = lax.bitcast_convert_type(eb[2, pl.ds(g * 16, 16)], jnp.float32)
            for i in range(16):
                wv = lax.gather(
                    w16, jnp.full((16, 1), i, jnp.int32),
                    dimension_numbers=lax.GatherDimensionNumbers(
                        offset_dims=(), collapsed_slice_dims=(0,),
                        start_index_map=(0,)),
                    slice_sizes=(1,),
                    mode=lax.GatherScatterMode.PROMISE_IN_BOUNDS)
                for j in range(_LG):
                    sl = pl.ds(j * 16, 16)
                    rb[g * 16 + i, sl] = rb[g * 16 + i, sl] * wv
            return carry
        lax.fori_loop(0, _C // 16, _e16, 0)

    for t in range(6):
        _fire_edata(t, t)
    _wait_edata(0, 0)
    _gather(0, 0, 0)
    _wait_edata(1, 1)
    _gather(1, 1, 1)

    def _substep(t, b, e):
        bf = (b + 2) % 4

        @pl.when(t + 2 < _NCHUNK)
        def _():
            @pl.when(t >= 2)
            def _():
                _wait_scatter(t - 2, bf, (e + 6) % 8)

            @pl.when(t + 6 < _NCHUNK)
            def _():
                _fire_edata(t + 6, (e + 6) % 8)
            _wait_edata(t + 2, (e + 2) % 8)
            _gather(t + 2, bf, (e + 2) % 8)
        _wait_gather(t, b, e)
        _scale(b, e)
        _scatter(t, b, e)

    def _outer(i, carry):
        for b8 in range(8):
            _substep(i * 8 + b8, b8 % 4, b8)
        return carry
    lax.fori_loop(0, _NCHUNK // 8, _outer, 0)

    for t in range((_NCHUNK // 8) * 8, _NCHUNK):
        _substep(t, t % 4, t % 8)
    for t in range(_NCHUNK - 4, _NCHUNK):
        _wait_scatter(t, t % 4, t % 8)

    plsc.subcore_barrier()
    for rep in range(-(-_NCHUNK // 16)):
        cid = s + 16 * rep

        @pl.when(cid < _NCHUNK)
        def _():
            o = cid * _C
            pltpu.sync_copy(acc_sh.at[pl.ds(o, _C)], out_hbm.at[c, pl.ds(o, _C)])


@functools.lru_cache(maxsize=None)
def _make_spmv(combine):
    scratch = (
        [pltpu.VMEM((3, _C), jnp.int32) for _ in range(8)]
        + [pltpu.VMEM((_C, _F), jnp.float32) for _ in range(4)]
        + [pltpu.VMEM_SHARED((_M, _F), jnp.float32)]
        + [pltpu.SemaphoreType.DMA] * 16
    )
    part = jax.ShapeDtypeStruct((2, _M, _F), jnp.float32)
    comb = jax.ShapeDtypeStruct((_M, _F), jnp.float32)
    return pl.kernel(
        functools.partial(_spmv_body, combine),
        out_type=(part, comb) if combine else part,
        scratch_types=scratch,
        mesh=plsc.VectorSubcoreMesh(core_axis_name="c", subcore_axis_name="s"),
    )


def _combine_body(p_ref, o_ref):
    o_ref[...] = p_ref[0] + p_ref[1]


def _combine(p):
    return pl.pallas_call(
        _combine_body,
        grid=(10,),
        in_specs=[pl.BlockSpec((2, _M // 10, _F), lambda i: (0, i, 0))],
        out_specs=pl.BlockSpec((_M // 10, _F), lambda i: (i, 0)),
        out_shape=jax.ShapeDtypeStruct((_M, _F), jnp.float32),
    )(p)


def _dense_body(x0_ref, s1_ref, s2_ref, p3_ref, dk_ref, pk_ref, o_ref):
    dk = dk_ref[...]
    pkt = pk_ref[...].T
    w0 = pkt * dk[:, 0, 0][:, None]
    w1 = pkt * dk[:, 0, 1][:, None]
    w2 = pkt * dk[:, 0, 2][:, None]
    w3 = pkt * dk[:, 0, 3][:, None]
    v0 = w0 - w2
    v1 = w1 - 3.0 * w3
    v2 = 2.0 * w2
    v3 = 4.0 * w3
    s3 = p3_ref[0] + p3_ref[1]
    acc = jnp.dot(x0_ref[...], v0, preferred_element_type=jnp.float32)
    acc += jnp.dot(s1_ref[...], v1, preferred_element_type=jnp.float32)
    acc += jnp.dot(s2_ref[...], v2, preferred_element_type=jnp.float32)
    acc += jnp.dot(s3, v3, preferred_element_type=jnp.float32)
    o_ref[...] = acc


def _dense(x0, s1, s2, p3, dkernel, pkernel):
    bm = _M // 10
    return pl.pallas_call(
        _dense_body,
        grid=(10,),
        in_specs=[
            pl.BlockSpec((bm, _F), lambda i: (i, 0)),
            pl.BlockSpec((bm, _F), lambda i: (i, 0)),
            pl.BlockSpec((bm, _F), lambda i: (i, 0)),
            pl.BlockSpec((2, bm, _F), lambda i: (0, i, 0)),
            pl.BlockSpec((_F, 1, 4), lambda i: (0, 0, 0)),
            pl.BlockSpec((_F, _F), lambda i: (0, 0)),
        ],
        out_specs=pl.BlockSpec((bm, _F), lambda i: (i, 0)),
        out_shape=jax.ShapeDtypeStruct((_M, _F), jnp.float32),
    )(x0, s1, s2, p3, dkernel, pkernel)


def kernel(x, edge_index, edge_weight, dkernel, pkernel):
    m = x.shape[1]
    x0 = jnp.transpose(x, (1, 0, 2)).reshape(m, -1)
    src = edge_index[0].astype(jnp.int32).reshape(_NW, _NCHUNK, _C)
    dst = edge_index[1].astype(jnp.int32).reshape(_NW, _NCHUNK, _C)
    wbits = lax.bitcast_convert_type(
        edge_weight.astype(jnp.float32), jnp.int32).reshape(_NW, _NCHUNK, _C)
    ed = jnp.stack([src, dst, wbits], axis=2)

    spmv0 = _make_spmv(False)
    spmvc = _make_spmv(True)
    p1 = spmv0(x0, ed)
    p2, s1 = spmvc(p1, ed)
    p3, s2 = spmvc(p2, ed)

    out = _dense(x0, s1, s2, p3, dkernel, pkernel)
    return out.reshape(1, m, -1)


# speedup vs baseline: 2.6643x; 1.1132x over previous
import functools

import jax
import jax.numpy as jnp
from jax import lax
from jax.experimental import pallas as pl
from jax.experimental.pallas import tpu as pltpu
from jax.experimental.pallas import tpu_sc as plsc

_M = 10000
_F = 128
_E = 320000
_NW = 32
_EPW = _E // _NW
_C = 80
_NCHUNK = _EPW // _C
_RC = 200
_NRC = _M // _RC
_RREP = -(-_NRC // 16)
_LG = _F // 16


def _spmv_body(h_hbm, ed_hbm, out_hbm, *refs):
    ed = refs[0:8]
    rows = refs[8:12]
    acc_sh = refs[12]
    sem_g = refs[13:17]
    sem_s = refs[17:21]
    sem_e = refs[21:29]
    c = lax.axis_index("c")
    s = lax.axis_index("s")
    wid = c * 16 + s

    r0 = rows[0]

    def _zrow(r, carry):
        for j in range(_LG):
            r0[r, pl.ds(j * 16, 16)] = jnp.zeros((16,), jnp.float32)
        return carry
    lax.fori_loop(0, _C, _zrow, 0)
    for rep in range(-(-_NCHUNK // 16)):
        cid = s + 16 * rep

        @pl.when(cid < _NCHUNK)
        def _():
            pltpu.sync_copy(r0, acc_sh.at[pl.ds(cid * _C, _C)])
    plsc.subcore_barrier()

    def _fire_edata(t, e):
        pltpu.async_copy(ed_hbm.at[wid, t], ed[e], sem_e[e])

    def _wait_edata(t, e):
        pltpu.make_async_copy(ed_hbm.at[wid, t], ed[e], sem_e[e]).wait()

    def _gather(t, b, e):
        pltpu.async_copy(h_hbm.at[ed[e].at[0]], rows[b], sem_g[b])

    def _wait_gather(t, b, e):
        pltpu.make_async_copy(h_hbm.at[ed[e].at[0]], rows[b], sem_g[b]).wait()

    def _scatter(t, b, e):
        pltpu.async_copy(rows[b], acc_sh.at[ed[e].at[1]], sem_s[b], add=True, priority=1)

    def _wait_scatter(t, b, e):
        pltpu.make_async_copy(rows[b], acc_sh.at[ed[e].at[1]], sem_s[b]).wait()

    def _scale(b, e):
        rb = rows[b]
        eb = ed[e]

        def _e16(g, carry):
            w16 = lax.bitcast_convert_type(eb[2, pl.ds(g * 16, 16)], jnp.float32)
            for i in range(16):
                wv = lax.gather(
                    w16, jnp.full((16, 1), i, jnp.int32),
                    dimension_numbers=lax.GatherDimensionNumbers(
                        offset_dims=(), collapsed_slice_dims=(0,),
                        start_index_map=(0,)),
                    slice_sizes=(1,),
                    mode=lax.GatherScatterMode.PROMISE_IN_BOUNDS)
                for j in range(_LG):
                    sl = pl.ds(j * 16, 16)
                    rb[g * 16 + i, sl] = rb[g * 16 + i, sl] * wv
            return carry
        lax.fori_loop(0, _C // 16, _e16, 0)

    for t in range(6):
        _fire_edata(t, t)
    _wait_edata(0, 0)
    _gather(0, 0, 0)
    _wait_edata(1, 1)
    _gather(1, 1, 1)

    def _substep(t, b, e):
        bf = (b + 2) % 4

        @pl.when(t + 2 < _NCHUNK)
        def _():
            @pl.when(t >= 2)
            def _():
                _wait_scatter(t - 2, bf, (e + 6) % 8)

            @pl.when(t + 6 < _NCHUNK)
            def _():
                _fire_edata(t + 6, (e + 6) % 8)
            _wait_edata(t + 2, (e + 2) % 8)
            _gather(t + 2, bf, (e + 2) % 8)
        _wait_gather(t, b, e)
        _scale(b, e)
        _scatter(t, b, e)

    def _outer(i, carry):
        for b8 in range(8):
            _substep(i * 8 + b8, b8 % 4, b8)
        return carry
    lax.fori_loop(0, _NCHUNK // 8, _outer, 0)

    for t in range((_NCHUNK // 8) * 8, _NCHUNK):
        _substep(t, t % 4, t % 8)
    for t in range(_NCHUNK - 4, _NCHUNK):
        _wait_scatter(t, t % 4, t % 8)

    plsc.subcore_barrier()
    for rep in range(-(-_NCHUNK // 16)):
        cid = s + 16 * rep

        @pl.when(cid < _NCHUNK)
        def _():
            o = cid * _C
            pltpu.sync_copy(acc_sh.at[pl.ds(o, _C)], out_hbm.at[c, pl.ds(o, _C)])


@functools.lru_cache(maxsize=None)
def _make_spmv():
    scratch = (
        [pltpu.VMEM((3, _C), jnp.int32) for _ in range(8)]
        + [pltpu.VMEM((_C, _F), jnp.float32) for _ in range(4)]
        + [pltpu.VMEM_SHARED((_M, _F), jnp.float32)]
        + [pltpu.SemaphoreType.DMA] * 16
    )
    return pl.kernel(
        _spmv_body,
        out_type=jax.ShapeDtypeStruct((2, _M, _F), jnp.float32),
        scratch_types=scratch,
        mesh=plsc.VectorSubcoreMesh(core_axis_name="c", subcore_axis_name="s"),
    )


def _combine_body(p_ref, o_ref):
    o_ref[...] = p_ref[0] + p_ref[1]


def _combine(p):
    return pl.pallas_call(
        _combine_body,
        grid=(10,),
        in_specs=[pl.BlockSpec((2, _M // 10, _F), lambda i: (0, i, 0))],
        out_specs=pl.BlockSpec((_M // 10, _F), lambda i: (i, 0)),
        out_shape=jax.ShapeDtypeStruct((_M, _F), jnp.float32),
    )(p)


def _dense_body(x0_ref, s1_ref, s2_ref, p3_ref, dk_ref, pk_ref, o_ref):
    dk = dk_ref[...]
    pkt = pk_ref[...].T
    w0 = pkt * dk[:, 0, 0][:, None]
    w1 = pkt * dk[:, 0, 1][:, None]
    w2 = pkt * dk[:, 0, 2][:, None]
    w3 = pkt * dk[:, 0, 3][:, None]
    v0 = w0 - w2
    v1 = w1 - 3.0 * w3
    v2 = 2.0 * w2
    v3 = 4.0 * w3
    s3 = p3_ref[0] + p3_ref[1]
    acc = jnp.dot(x0_ref[...], v0, preferred_element_type=jnp.float32)
    acc += jnp.dot(s1_ref[...], v1, preferred_element_type=jnp.float32)
    acc += jnp.dot(s2_ref[...], v2, preferred_element_type=jnp.float32)
    acc += jnp.dot(s3, v3, preferred_element_type=jnp.float32)
    o_ref[...] = acc


def _dense(x0, s1, s2, p3, dkernel, pkernel):
    bm = _M // 10
    return pl.pallas_call(
        _dense_body,
        grid=(10,),
        in_specs=[
            pl.BlockSpec((bm, _F), lambda i: (i, 0)),
            pl.BlockSpec((bm, _F), lambda i: (i, 0)),
            pl.BlockSpec((bm, _F), lambda i: (i, 0)),
            pl.BlockSpec((2, bm, _F), lambda i: (0, i, 0)),
            pl.BlockSpec((_F, 1, 4), lambda i: (0, 0, 0)),
            pl.BlockSpec((_F, _F), lambda i: (0, 0)),
        ],
        out_specs=pl.BlockSpec((bm, _F), lambda i: (i, 0)),
        out_shape=jax.ShapeDtypeStruct((_M, _F), jnp.float32),
    )(x0, s1, s2, p3, dkernel, pkernel)


def kernel(x, edge_index, edge_weight, dkernel, pkernel):
    m = x.shape[1]
    x0 = jnp.transpose(x, (1, 0, 2)).reshape(m, -1)
    src = edge_index[0].astype(jnp.int32).reshape(_NW, _NCHUNK, _C)
    dst = edge_index[1].astype(jnp.int32).reshape(_NW, _NCHUNK, _C)
    wbits = lax.bitcast_convert_type(
        edge_weight.astype(jnp.float32), jnp.int32).reshape(_NW, _NCHUNK, _C)
    ed = jnp.stack([src, dst, wbits], axis=2)

    spmv = _make_spmv()
    p1 = spmv(x0, ed)
    s1 = _combine(p1)
    p2 = spmv(s1, ed)
    s2 = _combine(p2)
    p3 = spmv(s2, ed)

    out = _dense(x0, s1, s2, p3, dkernel, pkernel)
    return out.reshape(1, m, -1)


# speedup vs baseline: 2.9818x; 1.1192x over previous
import functools

import jax
import jax.numpy as jnp
from jax import lax
from jax.experimental import pallas as pl
from jax.experimental.pallas import tpu as pltpu
from jax.experimental.pallas import tpu_sc as plsc

_M = 10000
_F = 128
_E = 320000
_NW = 32
_EPW = _E // _NW
_C = 80
_NCHUNK = _EPW // _C
_RC = 200
_NRC = _M // _RC
_RREP = -(-_NRC // 16)
_LG = _F // 16


def _spmv_body(h_hbm, ed_hbm, out_hbm, *refs):
    ed = refs[0:8]
    rows = refs[8:12]
    acc_sh = refs[12]
    sem_g = refs[13:17]
    sem_s = refs[17:21]
    sem_e = refs[21:29]
    c = lax.axis_index("c")
    s = lax.axis_index("s")
    wid = c * 16 + s

    r0 = rows[0]

    def _zrow(r, carry):
        for j in range(_LG):
            r0[r, pl.ds(j * 16, 16)] = jnp.zeros((16,), jnp.float32)
        return carry
    lax.fori_loop(0, _C, _zrow, 0)
    for rep in range(-(-_NCHUNK // 16)):
        cid = s + 16 * rep

        @pl.when(cid < _NCHUNK)
        def _():
            pltpu.sync_copy(r0, acc_sh.at[pl.ds(cid * _C, _C)])
    plsc.subcore_barrier()

    def _fire_edata(t, e):
        pltpu.async_copy(ed_hbm.at[wid, t], ed[e], sem_e[e])

    def _wait_edata(t, e):
        pltpu.make_async_copy(ed_hbm.at[wid, t], ed[e], sem_e[e]).wait()

    def _gather(t, b, e):
        pltpu.async_copy(h_hbm.at[ed[e].at[0]], rows[b], sem_g[b])

    def _wait_gather(t, b, e):
        pltpu.make_async_copy(h_hbm.at[ed[e].at[0]], rows[b], sem_g[b]).wait()

    def _scatter(t, b, e):
        pltpu.async_copy(rows[b], acc_sh.at[ed[e].at[1]], sem_s[b], add=True)

    def _wait_scatter(t, b, e):
        pltpu.make_async_copy(rows[b], acc_sh.at[ed[e].at[1]], sem_s[b]).wait()

    def _scale(b, e):
        rb = rows[b]
        eb = ed[e]

        @functools.partial(plsc.parallel_loop, 0, _C // 16, unroll=2)
        def _e16(g):
            w16 = lax.bitcast_convert_type(eb[2, pl.ds(g * 16, 16)], jnp.float32)
            for i in range(16):
                wv = lax.gather(
                    w16, jnp.full((16, 1), i, jnp.int32),
                    dimension_numbers=lax.GatherDimensionNumbers(
                        offset_dims=(), collapsed_slice_dims=(0,),
                        start_index_map=(0,)),
                    slice_sizes=(1,),
                    mode=lax.GatherScatterMode.PROMISE_IN_BOUNDS)
                for j in range(_LG):
                    sl = pl.ds(j * 16, 16)
                    rb[g * 16 + i, sl] = rb[g * 16 + i, sl] * wv

    for t in range(6):
        _fire_edata(t, t)
    _wait_edata(0, 0)
    _gather(0, 0, 0)
    _wait_edata(1, 1)
    _gather(1, 1, 1)

    def _substep(t, b, e):
        bf = (b + 2) % 4

        @pl.when(t + 2 < _NCHUNK)
        def _():
            @pl.when(t >= 2)
            def _():
                _wait_scatter(t - 2, bf, (e + 6) % 8)

            @pl.when(t + 6 < _NCHUNK)
            def _():
                _fire_edata(t + 6, (e + 6) % 8)
            _wait_edata(t + 2, (e + 2) % 8)
            _gather(t + 2, bf, (e + 2) % 8)
        _wait_gather(t, b, e)
        _scale(b, e)
        _scatter(t, b, e)

    def _outer(i, carry):
        for b8 in range(8):
            _substep(i * 8 + b8, b8 % 4, b8)
        return carry
    lax.fori_loop(0, _NCHUNK // 8, _outer, 0)

    for t in range((_NCHUNK // 8) * 8, _NCHUNK):
        _substep(t, t % 4, t % 8)
    for t in range(_NCHUNK - 4, _NCHUNK):
        _wait_scatter(t, t % 4, t % 8)

    plsc.subcore_barrier()
    for rep in range(-(-_NCHUNK // 16)):
        cid = s + 16 * rep

        @pl.when(cid < _NCHUNK)
        def _():
            o = cid * _C
            pltpu.sync_copy(acc_sh.at[pl.ds(o, _C)], out_hbm.at[c, pl.ds(o, _C)])


@functools.lru_cache(maxsize=None)
def _make_spmv():
    scratch = (
        [pltpu.VMEM((3, _C), jnp.int32) for _ in range(8)]
        + [pltpu.VMEM((_C, _F), jnp.float32) for _ in range(4)]
        + [pltpu.VMEM_SHARED((_M, _F), jnp.float32)]
        + [pltpu.SemaphoreType.DMA] * 16
    )
    return pl.kernel(
        _spmv_body,
        out_type=jax.ShapeDtypeStruct((2, _M, _F), jnp.float32),
        scratch_types=scratch,
        mesh=plsc.VectorSubcoreMesh(core_axis_name="c", subcore_axis_name="s"),
    )


def _combine_body(p_ref, o_ref):
    o_ref[...] = p_ref[0] + p_ref[1]


def _combine(p):
    return pl.pallas_call(
        _combine_body,
        grid=(10,),
        in_specs=[pl.BlockSpec((2, _M // 10, _F), lambda i: (0, i, 0))],
        out_specs=pl.BlockSpec((_M // 10, _F), lambda i: (i, 0)),
        out_shape=jax.ShapeDtypeStruct((_M, _F), jnp.float32),
    )(p)


def _dense_body(x0_ref, s1_ref, s2_ref, p3_ref, dk_ref, pk_ref, o_ref):
    dk = dk_ref[...]
    pkt = pk_ref[...].T
    w0 = pkt * dk[:, 0, 0][:, None]
    w1 = pkt * dk[:, 0, 1][:, None]
    w2 = pkt * dk[:, 0, 2][:, None]
    w3 = pkt * dk[:, 0, 3][:, None]
    v0 = w0 - w2
    v1 = w1 - 3.0 * w3
    v2 = 2.0 * w2
    v3 = 4.0 * w3
    s3 = p3_ref[0] + p3_ref[1]
    acc = jnp.dot(x0_ref[...], v0, preferred_element_type=jnp.float32)
    acc += jnp.dot(s1_ref[...], v1, preferred_element_type=jnp.float32)
    acc += jnp.dot(s2_ref[...], v2, preferred_element_type=jnp.float32)
    acc += jnp.dot(s3, v3, preferred_element_type=jnp.float32)
    o_ref[...] = acc


def _dense(x0, s1, s2, p3, dkernel, pkernel):
    bm = _M // 10
    return pl.pallas_call(
        _dense_body,
        grid=(10,),
        in_specs=[
            pl.BlockSpec((bm, _F), lambda i: (i, 0)),
            pl.BlockSpec((bm, _F), lambda i: (i, 0)),
            pl.BlockSpec((bm, _F), lambda i: (i, 0)),
            pl.BlockSpec((2, bm, _F), lambda i: (0, i, 0)),
            pl.BlockSpec((_F, 1, 4), lambda i: (0, 0, 0)),
            pl.BlockSpec((_F, _F), lambda i: (0, 0)),
        ],
        out_specs=pl.BlockSpec((bm, _F), lambda i: (i, 0)),
        out_shape=jax.ShapeDtypeStruct((_M, _F), jnp.float32),
    )(x0, s1, s2, p3, dkernel, pkernel)


def kernel(x, edge_index, edge_weight, dkernel, pkernel):
    m = x.shape[1]
    x0 = jnp.transpose(x, (1, 0, 2)).reshape(m, -1)
    src = edge_index[0].astype(jnp.int32).reshape(_NW, _NCHUNK, _C)
    dst = edge_index[1].astype(jnp.int32).reshape(_NW, _NCHUNK, _C)
    wbits = lax.bitcast_convert_type(
        edge_weight.astype(jnp.float32), jnp.int32).reshape(_NW, _NCHUNK, _C)
    ed = jnp.stack([src, dst, wbits], axis=2)

    spmv = _make_spmv()
    p1 = spmv(x0, ed)
    s1 = _combine(p1)
    p2 = spmv(s1, ed)
    s2 = _combine(p2)
    p3 = spmv(s2, ed)

    out = _dense(x0, s1, s2, p3, dkernel, pkernel)
    return out.reshape(1, m, -1)
